# baseline (speedup 1.0000x reference)
import functools
import jax
import jax.numpy as jnp
from jax import lax
from jax.experimental import pallas as pl
from jax.experimental.pallas import tpu as pltpu
from jax.experimental.pallas import tpu_sc as plsc

_LANES = 16
_STRIP = 16
_NBUF = 2
_UNROLL = 8
_N_WORKERS = 32


def _make_sc_add(batch, seq_len, d_model):
    n_strips = seq_len // _STRIP // _N_WORKERS
    slices_per_row = d_model // _LANES

    mesh = plsc.VectorSubcoreMesh(core_axis_name="c", subcore_axis_name="s")

    @functools.partial(
        pl.kernel,
        mesh=mesh,
        out_type=jax.ShapeDtypeStruct((batch, seq_len, d_model), jnp.float32),
        scratch_types=[
            pltpu.VMEM((_NBUF, _STRIP, d_model), jnp.float32),
            pltpu.VMEM((_STRIP, d_model), jnp.float32),
            pltpu.SemaphoreType.DMA((_NBUF,)),
            pltpu.SemaphoreType.DMA,
            pltpu.SemaphoreType.DMA((_NBUF,)),
        ],
    )
    def k(x_hbm, t_hbm, o_hbm, xbuf, tbuf, sx, st, so):
        wid = lax.axis_index("s") * 2 + lax.axis_index("c")

        def x_in(s0, b_i, slot):
            pltpu.make_async_copy(
                x_hbm.at[b_i, pl.ds(s0, _STRIP), :], xbuf.at[slot], sx.at[slot]
            ).start()

        def wait_x(slot):
            pltpu.make_async_copy(
                x_hbm.at[0, pl.ds(0, _STRIP), :], xbuf.at[slot], sx.at[slot]
            ).wait()

        def wait_out(slot):
            pltpu.make_async_copy(
                o_hbm.at[0, pl.ds(0, _STRIP), :], xbuf.at[slot], so.at[slot]
            ).wait()

        def strip_step(strip_i, carry0):
            s0 = (strip_i * _N_WORKERS + wid) * _STRIP
            pltpu.make_async_copy(
                t_hbm.at[pl.ds(s0, _STRIP), :], tbuf, st
            ).start()

            for b0 in range(_NBUF):
                x_in(s0, b0, b0)

            pltpu.make_async_copy(
                t_hbm.at[pl.ds(0, _STRIP), :], tbuf, st
            ).wait()

            def batch_group(gb, carry, _s0=s0):
                for sl in range(_NBUF):
                    b_i = gb * _NBUF + sl
                    wait_x(sl)

                    for r in range(_STRIP):

                        def grp_add(gg, c3, _sl=sl, _r=r):
                            for u in range(_UNROLL):
                                c0 = (gg * _UNROLL + u) * _LANES
                                plsc.addupdate(
                                    xbuf.at[_sl, _r, pl.ds(c0, _LANES)],
                                    tbuf[_r, pl.ds(c0, _LANES)],
                                )
                            return c3

                        lax.fori_loop(0, slices_per_row // _UNROLL, grp_add, 0)

                    pltpu.make_async_copy(
                        xbuf.at[sl], o_hbm.at[b_i, pl.ds(_s0, _STRIP), :], so.at[sl]
                    ).start()

                    @pl.when(b_i + _NBUF < batch)
                    def _(_sl=sl, _b=b_i):
                        wait_out(_sl)
                        x_in(_s0, _b + _NBUF, _sl)

                return carry

            lax.fori_loop(0, batch // _NBUF, batch_group, 0)

            for b0 in range(_NBUF):
                wait_out(b0)

            return carry0

        lax.fori_loop(0, n_strips, strip_step, 0)

    return k


def kernel(x, pos_table):
    batch, seq_len, d_model = x.shape
    table = pos_table[:seq_len]
    k = _make_sc_add(batch, seq_len, d_model)
    return k(x, table)


# speedup vs baseline: 1.1504x; 1.1504x over previous
import functools
import jax
import jax.numpy as jnp
from jax import lax
from jax.experimental import pallas as pl
from jax.experimental.pallas import tpu as pltpu
from jax.experimental.pallas import tpu_sc as plsc

_LANES = 16
_STRIP = 16
_NBUF = 4
_UNROLL = 8
_N_WORKERS = 32


def _make_sc_add(batch, seq_len, d_model):
    n_strips = seq_len // _STRIP // _N_WORKERS
    slices_per_row = d_model // _LANES

    mesh = plsc.VectorSubcoreMesh(core_axis_name="c", subcore_axis_name="s")

    @functools.partial(
        pl.kernel,
        mesh=mesh,
        out_type=jax.ShapeDtypeStruct((batch, seq_len, d_model), jnp.float32),
        scratch_types=[
            pltpu.VMEM((_NBUF, _STRIP, d_model), jnp.float32),
            pltpu.VMEM((_STRIP, d_model), jnp.float32),
            pltpu.SemaphoreType.DMA((_NBUF,)),
            pltpu.SemaphoreType.DMA,
            pltpu.SemaphoreType.DMA((_NBUF,)),
        ],
    )
    def k(x_hbm, t_hbm, o_hbm, xbuf, tbuf, sx, st, so):
        wid = lax.axis_index("s") * 2 + lax.axis_index("c")

        def x_in(s0, b_i, slot):
            pltpu.make_async_copy(
                x_hbm.at[b_i, pl.ds(s0, _STRIP), :], xbuf.at[slot], sx.at[slot]
            ).start()

        def wait_x(slot):
            pltpu.make_async_copy(
                x_hbm.at[0, pl.ds(0, _STRIP), :], xbuf.at[slot], sx.at[slot]
            ).wait()

        def wait_out(slot):
            pltpu.make_async_copy(
                o_hbm.at[0, pl.ds(0, _STRIP), :], xbuf.at[slot], so.at[slot]
            ).wait()

        def strip_step(strip_i, carry0):
            s0 = (strip_i * _N_WORKERS + wid) * _STRIP
            pltpu.make_async_copy(
                t_hbm.at[pl.ds(s0, _STRIP), :], tbuf, st
            ).start()

            for b0 in range(_NBUF):
                x_in(s0, b0, b0)

            pltpu.make_async_copy(
                t_hbm.at[pl.ds(0, _STRIP), :], tbuf, st
            ).wait()

            def batch_group(gb, carry, _s0=s0):
                for sl in range(_NBUF):
                    b_i = gb * _NBUF + sl
                    wait_x(sl)

                    for r in range(_STRIP):

                        def grp_add(gg, c3, _sl=sl, _r=r):
                            for u in range(_UNROLL):
                                c0 = (gg * _UNROLL + u) * _LANES
                                plsc.addupdate(
                                    xbuf.at[_sl, _r, pl.ds(c0, _LANES)],
                                    tbuf[_r, pl.ds(c0, _LANES)],
                                )
                            return c3

                        lax.fori_loop(0, slices_per_row // _UNROLL, grp_add, 0)

                    pltpu.make_async_copy(
                        xbuf.at[sl], o_hbm.at[b_i, pl.ds(_s0, _STRIP), :], so.at[sl]
                    ).start()

                    @pl.when(b_i + _NBUF < batch)
                    def _(_sl=sl, _b=b_i):
                        wait_out(_sl)
                        x_in(_s0, _b + _NBUF, _sl)

                return carry

            lax.fori_loop(0, batch // _NBUF, batch_group, 0)

            for b0 in range(_NBUF):
                wait_out(b0)

            return carry0

        lax.fori_loop(0, n_strips, strip_step, 0)

    return k


def kernel(x, pos_table):
    batch, seq_len, d_model = x.shape
    table = pos_table[:seq_len]
    k = _make_sc_add(batch, seq_len, d_model)
    return k(x, table)


# speedup vs baseline: 1.2617x; 1.0968x over previous
import functools
import jax
import jax.numpy as jnp
from jax import lax
from jax.experimental import pallas as pl
from jax.experimental.pallas import tpu as pltpu
from jax.experimental.pallas import tpu_sc as plsc

_LANES = 16
_CHUNK = 16
_NBUF = 3
_UNROLL = 8
_N_WORKERS = 32
_TC_BS = 256


def _tc_body(x_ref, p_ref, o_ref):
    o_ref[...] = x_ref[...] + p_ref[...]


def _make_sc_last_batch(batch, seq_len, d_model):
    rows_per_w = seq_len // _N_WORKERS
    n_chunks = rows_per_w // _CHUNK
    b_last = batch - 1

    mesh = plsc.VectorSubcoreMesh(core_axis_name="c", subcore_axis_name="s")

    @functools.partial(
        pl.kernel,
        mesh=mesh,
        out_type=jax.ShapeDtypeStruct((seq_len, d_model), jnp.float32),
        scratch_types=[
            pltpu.VMEM((_NBUF, _CHUNK, d_model), jnp.float32),
            pltpu.VMEM((_NBUF, _CHUNK, d_model), jnp.float32),
            pltpu.SemaphoreType.DMA((_NBUF,)),
            pltpu.SemaphoreType.DMA((_NBUF,)),
            pltpu.SemaphoreType.DMA((_NBUF,)),
        ],
    )
    def k(x_hbm, t_hbm, o_hbm, xbuf, tbuf, sx, st, so):
        wid = lax.axis_index("s") * 2 + lax.axis_index("c")
        s_base = wid * rows_per_w

        def start_in(ci, slot):
            s0 = s_base + ci * _CHUNK
            pltpu.make_async_copy(
                x_hbm.at[b_last, pl.ds(s0, _CHUNK), :], xbuf.at[slot], sx.at[slot]
            ).start()
            pltpu.make_async_copy(
                t_hbm.at[pl.ds(s0, _CHUNK), :], tbuf.at[slot], st.at[slot]
            ).start()

        def wait_in(slot):
            pltpu.make_async_copy(
                x_hbm.at[0, pl.ds(0, _CHUNK), :], xbuf.at[slot], sx.at[slot]
            ).wait()
            pltpu.make_async_copy(
                t_hbm.at[pl.ds(0, _CHUNK), :], tbuf.at[slot], st.at[slot]
            ).wait()

        def wait_out(slot):
            pltpu.make_async_copy(
                o_hbm.at[pl.ds(0, _CHUNK), :], xbuf.at[slot], so.at[slot]
            ).wait()

        for ci in range(min(_NBUF, n_chunks)):
            start_in(ci, ci)

        for ci in range(n_chunks):
            slot = ci % _NBUF
            wait_in(slot)

            for r in range(_CHUNK):

                def grp_add(gg, c3, _sl=slot, _r=r):
                    for u in range(_UNROLL):
                        c0 = (gg * _UNROLL + u) * _LANES
                        plsc.addupdate(
                            xbuf.at[_sl, _r, pl.ds(c0, _LANES)],
                            tbuf[_sl, _r, pl.ds(c0, _LANES)],
                        )
                    return c3

                lax.fori_loop(0, d_model // _LANES // _UNROLL, grp_add, 0)

            s0 = s_base + ci * _CHUNK
            pltpu.make_async_copy(
                xbuf.at[slot], o_hbm.at[pl.ds(s0, _CHUNK), :], so.at[slot]
            ).start()

            if ci + _NBUF < n_chunks:
                wait_out(slot)
                start_in(ci + _NBUF, slot)

        for ci in range(max(n_chunks - _NBUF, 0), n_chunks):
            wait_out(ci % _NBUF)

    return k


def kernel(x, pos_table):
    batch, seq_len, d_model = x.shape
    table = pos_table[:seq_len]

    tc_out = pl.pallas_call(
        _tc_body,
        grid=(seq_len // _TC_BS,),
        in_specs=[
            pl.BlockSpec((batch - 1, _TC_BS, d_model), lambda s: (0, s, 0)),
            pl.BlockSpec((_TC_BS, d_model), lambda s: (s, 0)),
        ],
        out_specs=pl.BlockSpec((batch - 1, _TC_BS, d_model), lambda s: (0, s, 0)),
        out_shape=jax.ShapeDtypeStruct((batch - 1, seq_len, d_model), x.dtype),
        compiler_params=pltpu.CompilerParams(
            dimension_semantics=("parallel",),
        ),
    )(x, table)

    sc_out = _make_sc_last_batch(batch, seq_len, d_model)(x, table)

    return jnp.concatenate([tc_out, sc_out[None]], axis=0)


# speedup vs baseline: 3.8426x; 3.0456x over previous
import jax
import jax.numpy as jnp
from jax.experimental import pallas as pl
from jax.experimental.pallas import tpu as pltpu

_BS = 256


def _add_body(x_ref, p_ref, o_ref):
    o_ref[...] = x_ref[...] + p_ref[...]


def kernel(x, pos_table):
    batch, seq_len, d_model = x.shape
    table = pos_table[:seq_len]
    return pl.pallas_call(
        _add_body,
        grid=(seq_len // _BS,),
        in_specs=[
            pl.BlockSpec((batch, _BS, d_model), lambda s: (0, s, 0)),
            pl.BlockSpec((_BS, d_model), lambda s: (s, 0)),
        ],
        out_specs=pl.BlockSpec((batch, _BS, d_model), lambda s: (0, s, 0)),
        out_shape=jax.ShapeDtypeStruct((batch, seq_len, d_model), x.dtype),
        compiler_params=pltpu.CompilerParams(
            dimension_semantics=("parallel",),
        ),
    )(x, table)
